# baseline (speedup 1.0000x reference)
import functools

import jax
import jax.numpy as jnp
from jax import lax
from jax.experimental import pallas as pl
from jax.experimental.pallas import tpu as pltpu
from jax.experimental.pallas import tpu_sc as plsc

N = 10000
E = 320000
D_IN = 128
H = 256
LATENT = 128
G = 64

CH = 128
E_PAD = 327680
NACC = 10112
ROWS_Z = 632
ROWS_W = 624

R = 1000
GRID = N // R


def _leaky(v):
    return jnp.where(v > 0, v, 0.2 * v)


@functools.lru_cache(maxsize=None)
def _make_sc_agg(edge_split: bool):
    n_chunks = (E_PAD // CH) // 32 if edge_split else (E_PAD // CH) // 16
    ept = n_chunks * CH
    mesh = plsc.VectorSubcoreMesh(core_axis_name="c", subcore_axis_name="s")

    @functools.partial(
        pl.kernel,
        out_type=(jax.ShapeDtypeStruct((N, 128), jnp.float32),
                  jax.ShapeDtypeStruct((N, 128), jnp.float32)),
        mesh=mesh,
        scratch_types=(
            [pltpu.VMEM((CH,), jnp.int32)] * 4 +
            [pltpu.VMEM((CH,), jnp.int32)] * 4 +
            [pltpu.VMEM((CH, 128), jnp.float32)] * 2 +
            [pltpu.VMEM_SHARED((NACC, 128), jnp.float32)] +
            [pltpu.SemaphoreType.DMA] * 6
        ),
    )
    def agg(t0, t1, src_h, dst_h, z_h, o0, o1,
            si0, si1, si2, si3, di0, di1, di2, di3, rb0, rb1, acc,
            mi0, mi1, mi2, mi3, mg0, mg1):
        sidx = (si0, si1, si2, si3)
        didx = (di0, di1, di2, di3)
        rowsb = (rb0, rb1)
        sem_i = (mi0, mi1, mi2, mi3)
        sem_g = (mg0, mg1)
        c = lax.axis_index("c")
        s = lax.axis_index("s")

        pltpu.sync_copy(z_h, rowsb[0])
        r0 = s * ROWS_Z
        for q in range(5):
            sz = 128 if q < 4 else ROWS_Z - 4 * 128
            pltpu.sync_copy(rowsb[0].at[pl.ds(0, sz)],
                            acc.at[pl.ds(r0 + q * 128, sz)])

        ebase = ((c * 16 + s) if edge_split else s) * ept

        def issue_idx(j, q):
            pltpu.async_copy(src_h.at[pl.ds(ebase + j * CH, CH)],
                             sidx[q], sem_i[q])
            pltpu.async_copy(dst_h.at[pl.ds(ebase + j * CH, CH)],
                             didx[q], sem_i[q])

        def wait_idx(q):
            pltpu.make_async_copy(src_h.at[pl.ds(0, CH)], sidx[q],
                                  sem_i[q]).wait()
            pltpu.make_async_copy(dst_h.at[pl.ds(0, CH)], didx[q],
                                  sem_i[q]).wait()

        def start_gather(q, r):
            @pl.when(c == 0)
            def _():
                pltpu.async_copy(t0.at[sidx[q]], rowsb[r], sem_g[r])

            @pl.when(c == 1)
            def _():
                pltpu.async_copy(t1.at[sidx[q]], rowsb[r], sem_g[r])

        def wait_gather(r):
            pltpu.make_async_copy(t0.at[sidx[0]], rowsb[r], sem_g[r]).wait()

        plsc.subcore_barrier()

        for q in range(4):
            issue_idx(q, q)
        wait_idx(0)
        start_gather(0, 0)

        nc = n_chunks

        def body(jo, carry):
            for b in range(4):
                j = jo * 4 + b
                r = b % 2

                @pl.when(j + 1 < nc)
                def _():
                    wait_idx((b + 1) % 4)
                    start_gather((b + 1) % 4, (b + 1) % 2)

                wait_gather(r)
                pltpu.sync_copy(rowsb[r], acc.at[didx[b]], add=True)

                @pl.when(j + 4 < nc)
                def _():
                    issue_idx(j + 4, b)
            return carry

        lax.fori_loop(0, nc // 4, body, 0)
        plsc.subcore_barrier()

        w0 = s * ROWS_W
        for q in range(5):
            sz = 128 if q < 4 else ROWS_W - 4 * 128
            rr = w0 + q * 128
            pltpu.sync_copy(acc.at[pl.ds(rr, sz)], rowsb[0].at[pl.ds(0, sz)])

            @pl.when(c == 0)
            def _():
                pltpu.sync_copy(rowsb[0].at[pl.ds(0, sz)], o0.at[pl.ds(rr, sz)])

            @pl.when(c == 1)
            def _():
                pltpu.sync_copy(rowsb[0].at[pl.ds(0, sz)], o1.at[pl.ds(rr, sz)])

        rem = N - 16 * ROWS_W
        rr = 16 * ROWS_W

        @pl.when(s == 15)
        def _():
            pltpu.sync_copy(acc.at[pl.ds(rr, rem)], rowsb[0].at[pl.ds(0, rem)])

            @pl.when(c == 0)
            def _():
                pltpu.sync_copy(rowsb[0].at[pl.ds(0, rem)], o0.at[pl.ds(rr, rem)])

            @pl.when(c == 1)
            def _():
                pltpu.sync_copy(rowsb[0].at[pl.ds(0, rem)], o1.at[pl.ds(rr, rem)])

    return agg


def _mlp1_l0_body(x_ref, p0_ref, p1_ref, w_ref, b_ref, a_ref, st_ref):
    h = x_ref[...] + p0_ref[...] + p1_ref[...]
    a = _leaky(jnp.dot(h, w_ref[...], preferred_element_type=jnp.float32)
               + b_ref[...])
    a_ref[...] = a
    s = jnp.sum(a, axis=0, keepdims=True)
    s2 = jnp.sum(a * a, axis=0, keepdims=True)
    upd = jnp.concatenate([s, s2], axis=0)
    i = pl.program_id(0)

    @pl.when(i == 0)
    def _():
        st_ref[...] = upd

    @pl.when(i != 0)
    def _():
        st_ref[...] = st_ref[...] + upd


def _mlp1_body(h0_ref, h1_ref, a0_ref, a1_ref, w_ref, b_ref, a_ref, st_ref):
    h = jnp.concatenate([h0_ref[...] + a0_ref[...],
                         h1_ref[...] + a1_ref[...]], axis=1)
    a = _leaky(jnp.dot(h, w_ref[...], preferred_element_type=jnp.float32)
               + b_ref[...])
    a_ref[...] = a
    s = jnp.sum(a, axis=0, keepdims=True)
    s2 = jnp.sum(a * a, axis=0, keepdims=True)
    upd = jnp.concatenate([s, s2], axis=0)
    i = pl.program_id(0)

    @pl.when(i == 0)
    def _():
        st_ref[...] = upd

    @pl.when(i != 0)
    def _():
        st_ref[...] = st_ref[...] + upd


def _mlp2_body(a_ref, st_ref, g_ref, be_ref, w_ref, b_ref, h0_ref, h1_ref):
    st = st_ref[...]
    mean = st[0:1] * (1.0 / N)
    var = st[1:2] * (1.0 / N) - mean * mean
    scale = g_ref[...] * lax.rsqrt(var + 1e-5)
    anorm = (a_ref[...] - mean) * scale + be_ref[...]
    hh = _leaky(jnp.dot(anorm, w_ref[...], preferred_element_type=jnp.float32)
                + b_ref[...])
    h0_ref[...] = hh[:, 0:128]
    h1_ref[...] = hh[:, 128:256]


def _pool_body(h0_ref, h1_ref, bt_ref, g_ref, b_ref, w_ref, fb_ref, out_ref,
               acc_ref):
    h = jnp.concatenate([h0_ref[...], h1_ref[...]], axis=1)
    oh = (lax.broadcasted_iota(jnp.int32, (G, R), 0) == bt_ref[0])
    part = jnp.dot(oh.astype(jnp.float32), h,
                   preferred_element_type=jnp.float32)
    i = pl.program_id(0)

    @pl.when(i == 0)
    def _():
        acc_ref[...] = part

    @pl.when(i != 0)
    def _():
        acc_ref[...] = acc_ref[...] + part

    @pl.when(i == GRID - 1)
    def _():
        pool = acc_ref[...]
        m = jnp.mean(pool, axis=0, keepdims=True)
        v = jnp.mean(pool * pool, axis=0, keepdims=True) - m * m
        yn = (pool - m) * lax.rsqrt(v + 1e-5) * g_ref[...] + b_ref[...]
        out_ref[...] = (jnp.dot(yn, w_ref[...],
                                preferred_element_type=jnp.float32)
                        + fb_ref[...])


def _row_spec(d):
    return pl.BlockSpec((R, d), lambda i: (i, 0))


def _fix_spec(shape):
    return pl.BlockSpec(shape, lambda i: (0, 0))


_tc_mlp1_l0 = pl.pallas_call(
    _mlp1_l0_body,
    grid=(GRID,),
    in_specs=[_row_spec(128), _row_spec(128), _row_spec(128),
              _fix_spec((D_IN, H)), _fix_spec((1, H))],
    out_specs=[_row_spec(H), _fix_spec((2, H))],
    out_shape=[jax.ShapeDtypeStruct((N, H), jnp.float32),
               jax.ShapeDtypeStruct((2, H), jnp.float32)],
)

_tc_mlp1 = pl.pallas_call(
    _mlp1_body,
    grid=(GRID,),
    in_specs=[_row_spec(128), _row_spec(128), _row_spec(128), _row_spec(128),
              _fix_spec((H, H)), _fix_spec((1, H))],
    out_specs=[_row_spec(H), _fix_spec((2, H))],
    out_shape=[jax.ShapeDtypeStruct((N, H), jnp.float32),
               jax.ShapeDtypeStruct((2, H), jnp.float32)],
)

_tc_mlp2 = pl.pallas_call(
    _mlp2_body,
    grid=(GRID,),
    in_specs=[_row_spec(H), _fix_spec((2, H)), _fix_spec((1, H)),
              _fix_spec((1, H)), _fix_spec((H, H)), _fix_spec((1, H))],
    out_specs=[_row_spec(128), _row_spec(128)],
    out_shape=[jax.ShapeDtypeStruct((N, 128), jnp.float32),
               jax.ShapeDtypeStruct((N, 128), jnp.float32)],
)

_tc_pool = pl.pallas_call(
    _pool_body,
    grid=(GRID,),
    in_specs=[_row_spec(128), _row_spec(128),
              pl.BlockSpec((1, 1, R), lambda i: (i, 0, 0)),
              _fix_spec((1, H)), _fix_spec((1, H)),
              _fix_spec((H, LATENT)), _fix_spec((1, LATENT))],
    out_specs=_fix_spec((G, LATENT)),
    out_shape=jax.ShapeDtypeStruct((G, LATENT), jnp.float32),
    scratch_shapes=[pltpu.VMEM((G, H), jnp.float32)],
)


def kernel(x, c0_W1, c0_b1, c0_g1, c0_be1, c0_W2, c0_b2,
           c1_W1, c1_b1, c1_g1, c1_be1, c1_W2, c1_b2,
           c2_W1, c2_b1, c2_g1, c2_be1, c2_W2, c2_b2,
           bn_g, bn_b, fc_W, fc_b, edge_index, batch):
    src = edge_index[0].astype(jnp.int32)
    dst = edge_index[1].astype(jnp.int32)
    pad = E_PAD - E
    src_p = jnp.concatenate([src, jnp.zeros((pad,), jnp.int32)])
    dst_p = jnp.concatenate([dst, jnp.full((pad,), N, jnp.int32)])
    zrows = jnp.zeros((128, 128), jnp.float32)

    zero1 = lax.optimization_barrier(jnp.zeros((1, 1), jnp.float32))
    x2 = x + zero1
    p0, p1 = _make_sc_agg(True)(x, x2, src_p, dst_p, zrows)
    a, st = _tc_mlp1_l0(x, p0, p1, c0_W1, c0_b1.reshape(1, H))
    h0, h1 = _tc_mlp2(a, st, c0_g1.reshape(1, H), c0_be1.reshape(1, H),
                      c0_W2, c0_b2.reshape(1, H))

    for (W1, b1, g1, be1, W2, b2) in (
            (c1_W1, c1_b1, c1_g1, c1_be1, c1_W2, c1_b2),
            (c2_W1, c2_b1, c2_g1, c2_be1, c2_W2, c2_b2)):
        g0h, g1h = _make_sc_agg(False)(h0, h1, src_p, dst_p, zrows)
        a, st = _tc_mlp1(h0, h1, g0h, g1h, W1, b1.reshape(1, H))
        h0, h1 = _tc_mlp2(a, st, g1.reshape(1, H), be1.reshape(1, H),
                          W2, b2.reshape(1, H))

    out = _tc_pool(h0, h1, batch.reshape(GRID, 1, R).astype(jnp.int32),
                   bn_g.reshape(1, H), bn_b.reshape(1, H),
                   fc_W, fc_b.reshape(1, LATENT))
    return out


# speedup vs baseline: 1.0852x; 1.0852x over previous
import functools

import jax
import jax.numpy as jnp
from jax import lax
from jax.experimental import pallas as pl
from jax.experimental.pallas import tpu as pltpu
from jax.experimental.pallas import tpu_sc as plsc

N = 10000
E = 320000
D_IN = 128
H = 256
LATENT = 128
G = 64

CH = 128
E_PAD = 327680
NACC = 10112
ROWS_Z = 632
ROWS_W = 624

R = 1000
GRID = N // R


def _leaky(v):
    return jnp.where(v > 0, v, 0.2 * v)


@functools.lru_cache(maxsize=None)
def _make_sc_agg(edge_split: bool):
    n_chunks = (E_PAD // CH) // 32 if edge_split else (E_PAD // CH) // 16
    ept = n_chunks * CH
    mesh = plsc.VectorSubcoreMesh(core_axis_name="c", subcore_axis_name="s")

    @functools.partial(
        pl.kernel,
        out_type=(jax.ShapeDtypeStruct((N, 128), jnp.float32),
                  jax.ShapeDtypeStruct((N, 128), jnp.float32)),
        mesh=mesh,
        scratch_types=(
            [pltpu.VMEM((CH,), jnp.int32)] * 4 +
            [pltpu.VMEM((CH,), jnp.int32)] * 4 +
            [pltpu.VMEM((CH, 128), jnp.float32)] * 2 +
            [pltpu.VMEM_SHARED((NACC, 128), jnp.float32)] +
            [pltpu.SemaphoreType.DMA] * 6
        ),
    )
    def agg(t0, t1, src_h, dst_h, z_h, o0, o1,
            si0, si1, si2, si3, di0, di1, di2, di3, rb0, rb1, acc,
            mi0, mi1, mi2, mi3, mg0, mg1):
        sidx = (si0, si1, si2, si3)
        didx = (di0, di1, di2, di3)
        rowsb = (rb0, rb1)
        sem_i = (mi0, mi1, mi2, mi3)
        sem_g = (mg0, mg1)
        c = lax.axis_index("c")
        s = lax.axis_index("s")

        pltpu.sync_copy(z_h, rowsb[0])
        r0 = s * ROWS_Z
        for q in range(5):
            sz = 128 if q < 4 else ROWS_Z - 4 * 128
            pltpu.sync_copy(rowsb[0].at[pl.ds(0, sz)],
                            acc.at[pl.ds(r0 + q * 128, sz)])

        ebase = ((c * 16 + s) if edge_split else s) * ept

        def issue_idx(j, q):
            pltpu.async_copy(src_h.at[pl.ds(ebase + j * CH, CH)],
                             sidx[q], sem_i[q])
            pltpu.async_copy(dst_h.at[pl.ds(ebase + j * CH, CH)],
                             didx[q], sem_i[q])

        def wait_idx(q):
            pltpu.make_async_copy(src_h.at[pl.ds(0, CH)], sidx[q],
                                  sem_i[q]).wait()
            pltpu.make_async_copy(dst_h.at[pl.ds(0, CH)], didx[q],
                                  sem_i[q]).wait()

        def start_gather(q, r):
            @pl.when(c == 0)
            def _():
                pltpu.async_copy(t0.at[sidx[q]], rowsb[r], sem_g[r])

            @pl.when(c == 1)
            def _():
                pltpu.async_copy(t1.at[sidx[q]], rowsb[r], sem_g[r])

        def wait_gather(r):
            pltpu.make_async_copy(t0.at[sidx[0]], rowsb[r], sem_g[r]).wait()

        plsc.subcore_barrier()

        for q in range(4):
            issue_idx(q, q)
        wait_idx(0)
        start_gather(0, 0)

        nc = n_chunks

        def body(jo, carry):
            for b in range(4):
                j = jo * 4 + b
                r = b % 2

                @pl.when(j + 1 < nc)
                def _():
                    wait_idx((b + 1) % 4)
                    start_gather((b + 1) % 4, (b + 1) % 2)

                wait_gather(r)
                pltpu.sync_copy(rowsb[r], acc.at[didx[b]], add=True)

                @pl.when(j + 4 < nc)
                def _():
                    issue_idx(j + 4, b)
            return carry

        lax.fori_loop(0, nc // 4, body, 0)
        plsc.subcore_barrier()

        w0 = s * ROWS_W
        for q in range(5):
            sz = 128 if q < 4 else ROWS_W - 4 * 128
            rr = w0 + q * 128
            pltpu.sync_copy(acc.at[pl.ds(rr, sz)], rowsb[0].at[pl.ds(0, sz)])

            @pl.when(c == 0)
            def _():
                pltpu.sync_copy(rowsb[0].at[pl.ds(0, sz)], o0.at[pl.ds(rr, sz)])

            @pl.when(c == 1)
            def _():
                pltpu.sync_copy(rowsb[0].at[pl.ds(0, sz)], o1.at[pl.ds(rr, sz)])

        rem = N - 16 * ROWS_W
        rr = 16 * ROWS_W

        @pl.when(s == 15)
        def _():
            pltpu.sync_copy(acc.at[pl.ds(rr, rem)], rowsb[0].at[pl.ds(0, rem)])

            @pl.when(c == 0)
            def _():
                pltpu.sync_copy(rowsb[0].at[pl.ds(0, rem)], o0.at[pl.ds(rr, rem)])

            @pl.when(c == 1)
            def _():
                pltpu.sync_copy(rowsb[0].at[pl.ds(0, rem)], o1.at[pl.ds(rr, rem)])

    return agg


def _mlp1_l0_body(x_ref, p0_ref, p1_ref, w_ref, b_ref, a_ref, st_ref):
    h = x_ref[...] + p0_ref[...] + p1_ref[...]
    a = _leaky(jnp.dot(h, w_ref[...], preferred_element_type=jnp.float32)
               + b_ref[...])
    a_ref[...] = a
    s = jnp.sum(a, axis=0, keepdims=True)
    s2 = jnp.sum(a * a, axis=0, keepdims=True)
    upd = jnp.concatenate([s, s2], axis=0)
    i = pl.program_id(0)

    @pl.when(i == 0)
    def _():
        st_ref[...] = upd

    @pl.when(i != 0)
    def _():
        st_ref[...] = st_ref[...] + upd


def _mlp1_body(h0_ref, h1_ref, a0_ref, a1_ref, w_ref, b_ref, a_ref, st_ref):
    h = jnp.concatenate([h0_ref[...] + a0_ref[...],
                         h1_ref[...] + a1_ref[...]], axis=1)
    a = _leaky(jnp.dot(h, w_ref[...], preferred_element_type=jnp.float32)
               + b_ref[...])
    a_ref[...] = a
    s = jnp.sum(a, axis=0, keepdims=True)
    s2 = jnp.sum(a * a, axis=0, keepdims=True)
    upd = jnp.concatenate([s, s2], axis=0)
    i = pl.program_id(0)

    @pl.when(i == 0)
    def _():
        st_ref[...] = upd

    @pl.when(i != 0)
    def _():
        st_ref[...] = st_ref[...] + upd


def _mlp2_body(a_ref, st_ref, g_ref, be_ref, w_ref, b_ref, h0_ref, h1_ref):
    st = st_ref[...]
    mean = st[0:1] * (1.0 / N)
    var = st[1:2] * (1.0 / N) - mean * mean
    scale = g_ref[...] * lax.rsqrt(var + 1e-5)
    anorm = (a_ref[...] - mean) * scale + be_ref[...]
    hh = _leaky(jnp.dot(anorm, w_ref[...], preferred_element_type=jnp.float32)
                + b_ref[...])
    h0_ref[...] = hh[:, 0:128]
    h1_ref[...] = hh[:, 128:256]


def _pool_body(h0_ref, h1_ref, bt_ref, g_ref, b_ref, w_ref, fb_ref, out_ref,
               acc_ref):
    h = jnp.concatenate([h0_ref[...], h1_ref[...]], axis=1)
    oh = (lax.broadcasted_iota(jnp.int32, (G, R), 0) == bt_ref[0])
    part = jnp.dot(oh.astype(jnp.float32), h,
                   preferred_element_type=jnp.float32)
    i = pl.program_id(0)

    @pl.when(i == 0)
    def _():
        acc_ref[...] = part

    @pl.when(i != 0)
    def _():
        acc_ref[...] = acc_ref[...] + part

    @pl.when(i == GRID - 1)
    def _():
        pool = acc_ref[...]
        m = jnp.mean(pool, axis=0, keepdims=True)
        v = jnp.mean(pool * pool, axis=0, keepdims=True) - m * m
        yn = (pool - m) * lax.rsqrt(v + 1e-5) * g_ref[...] + b_ref[...]
        out_ref[...] = (jnp.dot(yn, w_ref[...],
                                preferred_element_type=jnp.float32)
                        + fb_ref[...])


def _row_spec(d):
    return pl.BlockSpec((R, d), lambda i: (i, 0))


def _fix_spec(shape):
    return pl.BlockSpec(shape, lambda i: (0, 0))


_tc_mlp1_l0 = pl.pallas_call(
    _mlp1_l0_body,
    grid=(GRID,),
    in_specs=[_row_spec(128), _row_spec(128), _row_spec(128),
              _fix_spec((D_IN, H)), _fix_spec((1, H))],
    out_specs=[_row_spec(H), _fix_spec((2, H))],
    out_shape=[jax.ShapeDtypeStruct((N, H), jnp.float32),
               jax.ShapeDtypeStruct((2, H), jnp.float32)],
)

_tc_mlp1 = pl.pallas_call(
    _mlp1_body,
    grid=(GRID,),
    in_specs=[_row_spec(128), _row_spec(128), _row_spec(128), _row_spec(128),
              _fix_spec((H, H)), _fix_spec((1, H))],
    out_specs=[_row_spec(H), _fix_spec((2, H))],
    out_shape=[jax.ShapeDtypeStruct((N, H), jnp.float32),
               jax.ShapeDtypeStruct((2, H), jnp.float32)],
)

_tc_mlp2 = pl.pallas_call(
    _mlp2_body,
    grid=(GRID,),
    in_specs=[_row_spec(H), _fix_spec((2, H)), _fix_spec((1, H)),
              _fix_spec((1, H)), _fix_spec((H, H)), _fix_spec((1, H))],
    out_specs=[_row_spec(128), _row_spec(128)],
    out_shape=[jax.ShapeDtypeStruct((N, 128), jnp.float32),
               jax.ShapeDtypeStruct((N, 128), jnp.float32)],
)

_tc_pool = pl.pallas_call(
    _pool_body,
    grid=(GRID,),
    in_specs=[_row_spec(128), _row_spec(128),
              pl.BlockSpec((1, 1, R), lambda i: (i, 0, 0)),
              _fix_spec((1, H)), _fix_spec((1, H)),
              _fix_spec((H, LATENT)), _fix_spec((1, LATENT))],
    out_specs=_fix_spec((G, LATENT)),
    out_shape=jax.ShapeDtypeStruct((G, LATENT), jnp.float32),
    scratch_shapes=[pltpu.VMEM((G, H), jnp.float32)],
)


def kernel(x, c0_W1, c0_b1, c0_g1, c0_be1, c0_W2, c0_b2,
           c1_W1, c1_b1, c1_g1, c1_be1, c1_W2, c1_b2,
           c2_W1, c2_b1, c2_g1, c2_be1, c2_W2, c2_b2,
           bn_g, bn_b, fc_W, fc_b, edge_index, batch):
    src = edge_index[0].astype(jnp.int32)
    dst = edge_index[1].astype(jnp.int32)
    pad = E_PAD - E
    src_p = jnp.concatenate([src, jnp.zeros((pad,), jnp.int32)])
    dst_p = jnp.concatenate([dst, jnp.full((pad,), N, jnp.int32)])
    zrows = jnp.zeros((128, 128), jnp.float32)

    p0, p1 = _make_sc_agg(True)(x, x, src_p, dst_p, zrows)
    a, st = _tc_mlp1_l0(x, p0, p1, c0_W1, c0_b1.reshape(1, H))
    h0, h1 = _tc_mlp2(a, st, c0_g1.reshape(1, H), c0_be1.reshape(1, H),
                      c0_W2, c0_b2.reshape(1, H))

    for (W1, b1, g1, be1, W2, b2) in (
            (c1_W1, c1_b1, c1_g1, c1_be1, c1_W2, c1_b2),
            (c2_W1, c2_b1, c2_g1, c2_be1, c2_W2, c2_b2)):
        g0h, g1h = _make_sc_agg(False)(h0, h1, src_p, dst_p, zrows)
        a, st = _tc_mlp1(h0, h1, g0h, g1h, W1, b1.reshape(1, H))
        h0, h1 = _tc_mlp2(a, st, g1.reshape(1, H), be1.reshape(1, H),
                          W2, b2.reshape(1, H))

    out = _tc_pool(h0, h1, batch.reshape(GRID, 1, R).astype(jnp.int32),
                   bn_g.reshape(1, H), bn_b.reshape(1, H),
                   fc_W, fc_b.reshape(1, LATENT))
    return out
